# initial kernel scaffold (unmeasured)
import jax
import jax.numpy as jnp
from jax import lax
from jax.experimental import pallas as pl
from jax.experimental.pallas import tpu as pltpu

P = 4
MB = 1024
KB = 1024
NN = 2048


def kernel(x, w_mat):
    def body(x_hbm, w_hbm, out_ref, comm, wbuf, amax_tx, amax_rx,
             send_sems, recv_sems, a_send_sems, a_recv_sems, local_sems):
        me = lax.axis_index("i")

        bar = pltpu.get_barrier_semaphore()
        for off in (1, 2, 3):
            pl.semaphore_signal(bar, inc=1, device_id=((me + off) % P,),
                                device_id_type=pl.DeviceIdType.MESH)
        pl.semaphore_wait(bar, P - 1)

        sends = []
        for off in (1, 2, 3):
            t = (me + off) % P
            r = pltpu.make_async_remote_copy(
                src_ref=x_hbm.at[pl.ds(t * MB, MB), :],
                dst_ref=comm.at[3 - off],
                send_sem=send_sems.at[off - 1],
                recv_sem=recv_sems.at[3 - off],
                device_id=(t,),
                device_id_type=pl.DeviceIdType.MESH,
            )
            r.start()
            sends.append(r)

        own = pltpu.make_async_copy(
            x_hbm.at[pl.ds(me * MB, MB), :], comm.at[3], local_sems.at[0])
        own.start()
        w_dma = pltpu.make_async_copy(
            w_hbm.at[pl.ds(me * KB, KB), :], wbuf.at[0], local_sems.at[1])
        w_dma.start()
        own.wait()
        w_dma.wait()
        out_ref[...] = jnp.dot(comm[3], wbuf[0],
                               preferred_element_type=jnp.float32)

        for idx, j in enumerate((0, 2, 1)):
            s = (me + j + 1) % P
            wslot = (idx + 1) % 2
            w_dma = pltpu.make_async_copy(
                w_hbm.at[pl.ds(s * KB, KB), :], wbuf.at[wslot],
                local_sems.at[1])
            w_dma.start()
            recv = pltpu.make_async_remote_copy(
                src_ref=comm.at[j], dst_ref=comm.at[j],
                send_sem=send_sems.at[0], recv_sem=recv_sems.at[j],
                device_id=(me,), device_id_type=pl.DeviceIdType.MESH,
            )
            recv.wait_recv()
            w_dma.wait()
            out_ref[...] = out_ref[...] + jnp.dot(
                comm[j], wbuf[wslot], preferred_element_type=jnp.float32)

        for r in sends:
            r.wait_send()

        local_amax = jnp.maximum(jnp.max(out_ref[...]), 0.0)
        amax_tx[...] = jnp.broadcast_to(local_amax, (8, 128))
        a_sends = []
        for off in (1, 2, 3):
            t = (me + off) % P
            r = pltpu.make_async_remote_copy(
                src_ref=amax_tx, dst_ref=amax_rx.at[3 - off],
                send_sem=a_send_sems.at[off - 1],
                recv_sem=a_recv_sems.at[3 - off],
                device_id=(t,),
                device_id_type=pl.DeviceIdType.MESH,
            )
            r.start()
            a_sends.append(r)
        for j in (0, 1, 2):
            recv = pltpu.make_async_remote_copy(
                src_ref=amax_tx, dst_ref=amax_rx.at[j],
                send_sem=a_send_sems.at[0], recv_sem=a_recv_sems.at[j],
                device_id=(me,), device_id_type=pl.DeviceIdType.MESH,
            )
            recv.wait_recv()
        g_amax = jnp.maximum(local_amax, jnp.max(amax_rx[...]))

        scale = g_amax / 127.0
        y = jnp.maximum(out_ref[...], 0.0)
        q = jnp.clip(jnp.round(y / scale), -127.0, 127.0)
        out_ref[...] = q * scale

        for r in a_sends:
            r.wait_send()

    return pl.pallas_call(
        body,
        out_shape=jax.ShapeDtypeStruct((MB, NN), jnp.float32),
        in_specs=[
            pl.BlockSpec(memory_space=pltpu.ANY),
            pl.BlockSpec(memory_space=pltpu.ANY),
        ],
        out_specs=pl.BlockSpec(memory_space=pltpu.VMEM),
        scratch_shapes=[
            pltpu.VMEM((4, MB, KB), jnp.float32),
            pltpu.VMEM((2, KB, NN), jnp.float32),
            pltpu.VMEM((8, 128), jnp.float32),
            pltpu.VMEM((3, 8, 128), jnp.float32),
            pltpu.SemaphoreType.DMA((3,)),
            pltpu.SemaphoreType.DMA((3,)),
            pltpu.SemaphoreType.DMA((3,)),
            pltpu.SemaphoreType.DMA((3,)),
            pltpu.SemaphoreType.DMA((2,)),
        ],
        compiler_params=pltpu.CompilerParams(collective_id=0),
    )(x, w_mat)


# baseline (device time: 134210 ns/iter reference)
import jax
import jax.numpy as jnp
from jax import lax
from jax.experimental import pallas as pl
from jax.experimental.pallas import tpu as pltpu

P = 4
MB = 1024
KB = 1024
NN = 2048


def kernel(x, w_mat):
    def body(x_hbm, w_hbm, out_ref, comm, wbuf, amax_tx, amax_rx,
             send_sems, recv_sems, a_send_sems, a_recv_sems, local_sems):
        me = lax.axis_index("i")

        bar = pltpu.get_barrier_semaphore()
        for off in (1, 2, 3):
            pl.semaphore_signal(bar, inc=1, device_id=((me + off) % P,),
                                device_id_type=pl.DeviceIdType.MESH)
        pl.semaphore_wait(bar, P - 1)

        sends = []
        for off in (1, 2, 3):
            t = (me + off) % P
            r = pltpu.make_async_remote_copy(
                src_ref=x_hbm.at[pl.ds(t * MB, MB), :],
                dst_ref=comm.at[3 - off],
                send_sem=send_sems.at[off - 1],
                recv_sem=recv_sems.at[3 - off],
                device_id=(t,),
                device_id_type=pl.DeviceIdType.MESH,
            )
            r.start()
            sends.append(r)

        own = pltpu.make_async_copy(
            x_hbm.at[pl.ds(me * MB, MB), :], comm.at[3], local_sems.at[0])
        own.start()
        w_dma = pltpu.make_async_copy(
            w_hbm.at[pl.ds(me * KB, KB), :], wbuf.at[0], local_sems.at[1])
        w_dma.start()
        own.wait()
        w_dma.wait()
        out_ref[...] = jnp.dot(comm[3], wbuf[0],
                               preferred_element_type=jnp.float32)

        for idx, j in enumerate((0, 2, 1)):
            s = (me + j + 1) % P
            wslot = (idx + 1) % 2
            w_dma = pltpu.make_async_copy(
                w_hbm.at[pl.ds(s * KB, KB), :], wbuf.at[wslot],
                local_sems.at[1])
            w_dma.start()
            recv = pltpu.make_async_remote_copy(
                src_ref=comm.at[j], dst_ref=comm.at[j],
                send_sem=send_sems.at[0], recv_sem=recv_sems.at[j],
                device_id=(me,), device_id_type=pl.DeviceIdType.MESH,
            )
            recv.wait_recv()
            w_dma.wait()
            out_ref[...] = out_ref[...] + jnp.dot(
                comm[j], wbuf[wslot], preferred_element_type=jnp.float32)

        for r in sends:
            r.wait_send()

        local_amax = jnp.maximum(jnp.max(out_ref[...]), 0.0)
        amax_tx[...] = jnp.broadcast_to(local_amax, (8, 128))
        a_sends = []
        for off in (1, 2, 3):
            t = (me + off) % P
            r = pltpu.make_async_remote_copy(
                src_ref=amax_tx, dst_ref=amax_rx.at[3 - off],
                send_sem=a_send_sems.at[off - 1],
                recv_sem=a_recv_sems.at[3 - off],
                device_id=(t,),
                device_id_type=pl.DeviceIdType.MESH,
            )
            r.start()
            a_sends.append(r)
        for j in (0, 1, 2):
            recv = pltpu.make_async_remote_copy(
                src_ref=amax_tx, dst_ref=amax_rx.at[j],
                send_sem=a_send_sems.at[0], recv_sem=a_recv_sems.at[j],
                device_id=(me,), device_id_type=pl.DeviceIdType.MESH,
            )
            recv.wait_recv()
        g_amax = jnp.maximum(local_amax, jnp.max(amax_rx[...]))

        scale = g_amax / 127.0
        y = jnp.maximum(out_ref[...], 0.0)
        q = jnp.clip(jnp.round(y / scale), -127.0, 127.0)
        out_ref[...] = q * scale

        for r in a_sends:
            r.wait_send()

    return pl.pallas_call(
        body,
        out_shape=jax.ShapeDtypeStruct((MB, NN), jnp.float32),
        in_specs=[
            pl.BlockSpec(memory_space=pl.ANY),
            pl.BlockSpec(memory_space=pl.ANY),
        ],
        out_specs=pl.BlockSpec(memory_space=pltpu.VMEM),
        scratch_shapes=[
            pltpu.VMEM((4, MB, KB), jnp.float32),
            pltpu.VMEM((2, KB, NN), jnp.float32),
            pltpu.VMEM((8, 128), jnp.float32),
            pltpu.VMEM((3, 8, 128), jnp.float32),
            pltpu.SemaphoreType.DMA((3,)),
            pltpu.SemaphoreType.DMA((3,)),
            pltpu.SemaphoreType.DMA((3,)),
            pltpu.SemaphoreType.DMA((3,)),
            pltpu.SemaphoreType.DMA((2,)),
        ],
        compiler_params=pltpu.CompilerParams(
            collective_id=0,
            vmem_limit_bytes=52 * 1024 * 1024,
        ),
    )(x, w_mat)


# device time: 123910 ns/iter; 1.0831x vs baseline; 1.0831x over previous
import jax
import jax.numpy as jnp
from jax import lax
from jax.experimental import pallas as pl
from jax.experimental.pallas import tpu as pltpu

P = 4
MB = 1024
KB = 1024
NN = 2048
NC = 4
CH = MB // NC


def kernel(x, w_mat):
    def body(x_hbm, w_hbm, out_ref, comm, wbuf, amax_tx, amax_rx,
             send_sems, recv_sems, a_send_sems, a_recv_sems, local_sems):
        me = lax.axis_index("i")

        own = pltpu.make_async_copy(
            x_hbm.at[pl.ds(me * MB, MB), :], comm.at[3], local_sems.at[0])
        own.start()
        w0 = pltpu.make_async_copy(
            w_hbm.at[pl.ds(me * KB, KB), :], wbuf.at[0], local_sems.at[1])
        w0.start()
        s0 = (me + 1) % P
        w1 = pltpu.make_async_copy(
            w_hbm.at[pl.ds(s0 * KB, KB), :], wbuf.at[1], local_sems.at[2])
        w1.start()

        bar = pltpu.get_barrier_semaphore()
        for off in (1, 2, 3):
            pl.semaphore_signal(bar, inc=1, device_id=((me + off) % P,),
                                device_id_type=pl.DeviceIdType.MESH)
        pl.semaphore_wait(bar, P - 1)

        sends = []
        for h in range(NC):
            for off in (1, 2, 3):
                t = (me + off) % P
                r = pltpu.make_async_remote_copy(
                    src_ref=x_hbm.at[pl.ds(t * MB + h * CH, CH), :],
                    dst_ref=comm.at[3 - off, pl.ds(h * CH, CH), :],
                    send_sem=send_sems.at[off - 1, h],
                    recv_sem=recv_sems.at[3 - off, h],
                    device_id=(t,),
                    device_id_type=pl.DeviceIdType.MESH,
                )
                r.start()
                sends.append(r)

        def wait_chunk(j, h, src):
            recv = pltpu.make_async_remote_copy(
                src_ref=comm.at[j, pl.ds(h * CH, CH), :],
                dst_ref=comm.at[j, pl.ds(h * CH, CH), :],
                send_sem=send_sems.at[0, 0], recv_sem=recv_sems.at[j, h],
                device_id=(src,), device_id_type=pl.DeviceIdType.MESH,
            )
            recv.wait_recv()

        own.wait()
        w0.wait()
        out_ref[...] = jnp.dot(comm[3], wbuf[0],
                               preferred_element_type=jnp.float32)
        s2 = (me - 1) % P
        w_re0 = pltpu.make_async_copy(
            w_hbm.at[pl.ds(s2 * KB, KB), :], wbuf.at[0], local_sems.at[1])
        w_re0.start()

        w1.wait()
        for h in range(NC):
            wait_chunk(0, h, s0)
            rows = pl.ds(h * CH, CH)
            out_ref[rows, :] = out_ref[rows, :] + jnp.dot(
                comm[0, rows, :], wbuf[1], preferred_element_type=jnp.float32)
        s1 = (me + 2) % P
        w_re1 = pltpu.make_async_copy(
            w_hbm.at[pl.ds(s1 * KB, KB), :], wbuf.at[1], local_sems.at[2])
        w_re1.start()

        w_re0.wait()
        for h in range(NC):
            wait_chunk(2, h, s2)
            rows = pl.ds(h * CH, CH)
            out_ref[rows, :] = out_ref[rows, :] + jnp.dot(
                comm[2, rows, :], wbuf[0], preferred_element_type=jnp.float32)

        w_re1.wait()
        local_amax = jnp.float32(0.0)
        for h in range(NC):
            wait_chunk(1, h, s1)
            rows = pl.ds(h * CH, CH)
            chunk = jnp.maximum(
                out_ref[rows, :] + jnp.dot(comm[1, rows, :], wbuf[1],
                                           preferred_element_type=jnp.float32),
                0.0)
            out_ref[rows, :] = chunk
            local_amax = jnp.maximum(local_amax, jnp.max(chunk))

        amax_tx[...] = jnp.broadcast_to(local_amax, (8, 128))
        a_sends = []
        for off in (1, 2, 3):
            t = (me + off) % P
            r = pltpu.make_async_remote_copy(
                src_ref=amax_tx, dst_ref=amax_rx.at[3 - off],
                send_sem=a_send_sems.at[off - 1],
                recv_sem=a_recv_sems.at[3 - off],
                device_id=(t,),
                device_id_type=pl.DeviceIdType.MESH,
            )
            r.start()
            a_sends.append(r)
        for j in (0, 1, 2):
            recv = pltpu.make_async_remote_copy(
                src_ref=amax_tx, dst_ref=amax_rx.at[j],
                send_sem=a_send_sems.at[0], recv_sem=a_recv_sems.at[j],
                device_id=(me,), device_id_type=pl.DeviceIdType.MESH,
            )
            recv.wait_recv()
        g_amax = jnp.maximum(local_amax, jnp.max(amax_rx[...]))

        scale = g_amax / 127.0
        inv = 127.0 / g_amax
        q = jnp.clip(jnp.round(out_ref[...] * inv), 0.0, 127.0)
        out_ref[...] = q * scale

        for r in sends:
            r.wait_send()
        for r in a_sends:
            r.wait_send()

    return pl.pallas_call(
        body,
        out_shape=jax.ShapeDtypeStruct((MB, NN), jnp.float32),
        in_specs=[
            pl.BlockSpec(memory_space=pl.ANY),
            pl.BlockSpec(memory_space=pl.ANY),
        ],
        out_specs=pl.BlockSpec(memory_space=pltpu.VMEM),
        scratch_shapes=[
            pltpu.VMEM((4, MB, KB), jnp.float32),
            pltpu.VMEM((2, KB, NN), jnp.float32),
            pltpu.VMEM((8, 128), jnp.float32),
            pltpu.VMEM((3, 8, 128), jnp.float32),
            pltpu.SemaphoreType.DMA((3, NC)),
            pltpu.SemaphoreType.DMA((3, NC)),
            pltpu.SemaphoreType.DMA((3,)),
            pltpu.SemaphoreType.DMA((3,)),
            pltpu.SemaphoreType.DMA((3,)),
        ],
        compiler_params=pltpu.CompilerParams(
            collective_id=0,
            vmem_limit_bytes=52 * 1024 * 1024,
        ),
    )(x, w_mat)


# device time: 81880 ns/iter; 1.6391x vs baseline; 1.5133x over previous
import jax
import jax.numpy as jnp
from jax import lax
from jax.experimental import pallas as pl
from jax.experimental.pallas import tpu as pltpu

P = 4
MB = 1024
KB = 1024
NN = 2048
NC = 4
CH = MB // NC


def kernel(x, w_mat):
    def body(x_hbm, w_hbm, out_hbm, xstage, xbf, comm, wbuf, y_ref,
             amax_tx, amax_rx, send_sems, recv_sems, a_send_sems,
             a_recv_sems, local_sems, out_sems):
        me = lax.axis_index("i")

        stage = pltpu.make_async_copy(x_hbm, xstage, local_sems.at[0])
        stage.start()
        w0 = pltpu.make_async_copy(
            w_hbm.at[pl.ds(me * KB, KB), :], wbuf.at[0], local_sems.at[1])
        w0.start()
        s0 = (me + 1) % P
        w1 = pltpu.make_async_copy(
            w_hbm.at[pl.ds(s0 * KB, KB), :], wbuf.at[1], local_sems.at[2])
        w1.start()

        bar = pltpu.get_barrier_semaphore()
        for off in (1, 2, 3):
            pl.semaphore_signal(bar, inc=1, device_id=((me + off) % P,),
                                device_id_type=pl.DeviceIdType.MESH)
        pl.semaphore_wait(bar, P - 1)

        stage.wait()
        sends = []
        for off in (1, 3, 2):
            t = (me + off) % P
            xbf[off - 1] = xstage[pl.ds(t * MB, MB), :].astype(jnp.bfloat16)
            for h in range(NC):
                r = pltpu.make_async_remote_copy(
                    src_ref=xbf.at[off - 1, pl.ds(h * CH, CH), :],
                    dst_ref=comm.at[3 - off, pl.ds(h * CH, CH), :],
                    send_sem=send_sems.at[off - 1, h],
                    recv_sem=recv_sems.at[3 - off, h],
                    device_id=(t,),
                    device_id_type=pl.DeviceIdType.MESH,
                )
                r.start()
                sends.append(r)

        def wait_chunk(j, h, src):
            recv = pltpu.make_async_remote_copy(
                src_ref=comm.at[j, pl.ds(h * CH, CH), :],
                dst_ref=comm.at[j, pl.ds(h * CH, CH), :],
                send_sem=send_sems.at[0, 0], recv_sem=recv_sems.at[j, h],
                device_id=(src,), device_id_type=pl.DeviceIdType.MESH,
            )
            recv.wait_recv()

        w0.wait()
        y_ref[...] = jnp.dot(xstage[pl.ds(me * MB, MB), :], wbuf[0],
                             preferred_element_type=jnp.float32)
        s2 = (me - 1) % P
        w_re0 = pltpu.make_async_copy(
            w_hbm.at[pl.ds(s2 * KB, KB), :], wbuf.at[0], local_sems.at[1])
        w_re0.start()

        w1.wait()
        for h in range(NC):
            wait_chunk(0, h, s0)
            rows = pl.ds(h * CH, CH)
            out = jnp.dot(comm[0, rows, :].astype(jnp.float32), wbuf[1],
                          preferred_element_type=jnp.float32)
            y_ref[rows, :] = y_ref[rows, :] + out
        s1 = (me + 2) % P
        w_re1 = pltpu.make_async_copy(
            w_hbm.at[pl.ds(s1 * KB, KB), :], wbuf.at[1], local_sems.at[2])
        w_re1.start()

        w_re0.wait()
        for h in range(NC):
            wait_chunk(2, h, s2)
            rows = pl.ds(h * CH, CH)
            out = jnp.dot(comm[2, rows, :].astype(jnp.float32), wbuf[0],
                          preferred_element_type=jnp.float32)
            y_ref[rows, :] = y_ref[rows, :] + out

        w_re1.wait()
        local_amax = jnp.float32(0.0)
        for h in range(NC):
            wait_chunk(1, h, s1)
            rows = pl.ds(h * CH, CH)
            chunk = jnp.maximum(
                y_ref[rows, :] + jnp.dot(comm[1, rows, :].astype(jnp.float32),
                                         wbuf[1],
                                         preferred_element_type=jnp.float32),
                0.0)
            y_ref[rows, :] = chunk
            local_amax = jnp.maximum(local_amax, jnp.max(chunk))

        amax_tx[...] = jnp.broadcast_to(local_amax, (8, 128))
        a_sends = []
        for off in (1, 2, 3):
            t = (me + off) % P
            r = pltpu.make_async_remote_copy(
                src_ref=amax_tx, dst_ref=amax_rx.at[3 - off],
                send_sem=a_send_sems.at[off - 1],
                recv_sem=a_recv_sems.at[3 - off],
                device_id=(t,),
                device_id_type=pl.DeviceIdType.MESH,
            )
            r.start()
            a_sends.append(r)
        for j in (0, 1, 2):
            recv = pltpu.make_async_remote_copy(
                src_ref=amax_tx, dst_ref=amax_rx.at[j],
                send_sem=a_send_sems.at[0], recv_sem=a_recv_sems.at[j],
                device_id=(me,), device_id_type=pl.DeviceIdType.MESH,
            )
            recv.wait_recv()
        g_amax = jnp.maximum(local_amax, jnp.max(amax_rx[...]))

        scale = g_amax / 127.0
        inv = 127.0 / g_amax
        outs = []
        for h in range(NC):
            rows = pl.ds(h * CH, CH)
            q = jnp.clip(jnp.round(y_ref[rows, :] * inv), 0.0, 127.0)
            y_ref[rows, :] = q * scale
            cp = pltpu.make_async_copy(
                y_ref.at[rows, :], out_hbm.at[rows, :], out_sems.at[h])
            cp.start()
            outs.append(cp)
        for cp in outs:
            cp.wait()

        for r in sends:
            r.wait_send()
        for r in a_sends:
            r.wait_send()

    return pl.pallas_call(
        body,
        out_shape=jax.ShapeDtypeStruct((MB, NN), jnp.float32),
        in_specs=[
            pl.BlockSpec(memory_space=pl.ANY),
            pl.BlockSpec(memory_space=pl.ANY),
        ],
        out_specs=pl.BlockSpec(memory_space=pl.ANY),
        scratch_shapes=[
            pltpu.VMEM((P * MB, KB), jnp.float32),
            pltpu.VMEM((3, MB, KB), jnp.bfloat16),
            pltpu.VMEM((3, MB, KB), jnp.bfloat16),
            pltpu.VMEM((2, KB, NN), jnp.float32),
            pltpu.VMEM((MB, NN), jnp.float32),
            pltpu.VMEM((8, 128), jnp.float32),
            pltpu.VMEM((3, 8, 128), jnp.float32),
            pltpu.SemaphoreType.DMA((3, NC)),
            pltpu.SemaphoreType.DMA((3, NC)),
            pltpu.SemaphoreType.DMA((3,)),
            pltpu.SemaphoreType.DMA((3,)),
            pltpu.SemaphoreType.DMA((3,)),
            pltpu.SemaphoreType.DMA((NC,)),
        ],
        compiler_params=pltpu.CompilerParams(
            collective_id=0,
            vmem_limit_bytes=60 * 1024 * 1024,
        ),
    )(x, w_mat)


# device time: 79053 ns/iter; 1.6977x vs baseline; 1.0358x over previous
import jax
import jax.numpy as jnp
from jax import lax
from jax.experimental import pallas as pl
from jax.experimental.pallas import tpu as pltpu

P = 4
MB = 1024
KB = 1024
NN = 2048
NC = 4
CH = MB // NC
NQ = 8
CQ = MB // NQ


def kernel(x, w_mat):
    def body(x_hbm, w_hbm, out_hbm, xstage, xbf, comm, wbuf, y_ref,
             amax_tx, amax_rx, send_sems, recv_sems, a_send_sems,
             a_recv_sems, local_sems, out_sems):
        me = lax.axis_index("i")

        stage_dmas = []
        for off in (1, 3, 2):
            t = (me + off) % P
            d = pltpu.make_async_copy(
                x_hbm.at[pl.ds(t * MB, MB), :], xstage.at[off - 1],
                local_sems.at[off - 1])
            d.start()
            stage_dmas.append((off, t, d))
        own_dma = pltpu.make_async_copy(
            x_hbm.at[pl.ds(me * MB, MB), :], xstage.at[3], local_sems.at[3])
        own_dma.start()
        w0 = pltpu.make_async_copy(
            w_hbm.at[pl.ds(me * KB, KB), :], wbuf.at[0], local_sems.at[4])
        w0.start()
        s0 = (me + 1) % P
        w1 = pltpu.make_async_copy(
            w_hbm.at[pl.ds(s0 * KB, KB), :], wbuf.at[1], local_sems.at[5])
        w1.start()

        bar = pltpu.get_barrier_semaphore()
        for off in (1, 2, 3):
            pl.semaphore_signal(bar, inc=1, device_id=((me + off) % P,),
                                device_id_type=pl.DeviceIdType.MESH)
        pl.semaphore_wait(bar, P - 1)

        sends = []
        for off, t, d in stage_dmas:
            d.wait()
            xbf[off - 1] = xstage[off - 1].astype(jnp.bfloat16)
            for h in range(NC):
                r = pltpu.make_async_remote_copy(
                    src_ref=xbf.at[off - 1, pl.ds(h * CH, CH), :],
                    dst_ref=comm.at[3 - off, pl.ds(h * CH, CH), :],
                    send_sem=send_sems.at[off - 1, h],
                    recv_sem=recv_sems.at[3 - off, h],
                    device_id=(t,),
                    device_id_type=pl.DeviceIdType.MESH,
                )
                r.start()
                sends.append(r)

        def wait_chunk(j, h, src):
            recv = pltpu.make_async_remote_copy(
                src_ref=comm.at[j, pl.ds(h * CH, CH), :],
                dst_ref=comm.at[j, pl.ds(h * CH, CH), :],
                send_sem=send_sems.at[0, 0], recv_sem=recv_sems.at[j, h],
                device_id=(src,), device_id_type=pl.DeviceIdType.MESH,
            )
            recv.wait_recv()

        own_dma.wait()
        w0.wait()
        y_ref[...] = jnp.dot(xstage[3], wbuf[0],
                             preferred_element_type=jnp.float32)
        s2 = (me - 1) % P
        w_re0 = pltpu.make_async_copy(
            w_hbm.at[pl.ds(s2 * KB, KB), :], wbuf.at[0], local_sems.at[4])
        w_re0.start()

        w1.wait()
        for h in range(NC):
            wait_chunk(0, h, s0)
            rows = pl.ds(h * CH, CH)
            out = jnp.dot(comm[0, rows, :].astype(jnp.float32), wbuf[1],
                          preferred_element_type=jnp.float32)
            y_ref[rows, :] = y_ref[rows, :] + out
        s1 = (me + 2) % P
        w_re1 = pltpu.make_async_copy(
            w_hbm.at[pl.ds(s1 * KB, KB), :], wbuf.at[1], local_sems.at[5])
        w_re1.start()

        w_re0.wait()
        for h in range(NC):
            wait_chunk(2, h, s2)
            rows = pl.ds(h * CH, CH)
            out = jnp.dot(comm[2, rows, :].astype(jnp.float32), wbuf[0],
                          preferred_element_type=jnp.float32)
            y_ref[rows, :] = y_ref[rows, :] + out

        w_re1.wait()
        local_amax = jnp.float32(0.0)
        for h in range(NC):
            wait_chunk(1, h, s1)
            rows = pl.ds(h * CH, CH)
            chunk = jnp.maximum(
                y_ref[rows, :] + jnp.dot(comm[1, rows, :].astype(jnp.float32),
                                         wbuf[1],
                                         preferred_element_type=jnp.float32),
                0.0)
            y_ref[rows, :] = chunk
            local_amax = jnp.maximum(local_amax, jnp.max(chunk))

        amax_tx[...] = jnp.broadcast_to(local_amax, (8, 128))
        a_sends = []
        for off in (1, 2, 3):
            t = (me + off) % P
            r = pltpu.make_async_remote_copy(
                src_ref=amax_tx, dst_ref=amax_rx.at[3 - off],
                send_sem=a_send_sems.at[off - 1],
                recv_sem=a_recv_sems.at[3 - off],
                device_id=(t,),
                device_id_type=pl.DeviceIdType.MESH,
            )
            r.start()
            a_sends.append(r)
        for j in (0, 1, 2):
            recv = pltpu.make_async_remote_copy(
                src_ref=amax_tx, dst_ref=amax_rx.at[j],
                send_sem=a_send_sems.at[0], recv_sem=a_recv_sems.at[j],
                device_id=(me,), device_id_type=pl.DeviceIdType.MESH,
            )
            recv.wait_recv()
        g_amax = jnp.maximum(local_amax, jnp.max(amax_rx[...]))

        scale = g_amax / 127.0
        inv = 127.0 / g_amax
        outs = []
        for h in range(NQ):
            rows = pl.ds(h * CQ, CQ)
            y_ref[rows, :] = jnp.round(y_ref[rows, :] * inv) * scale
            cp = pltpu.make_async_copy(
                y_ref.at[rows, :], out_hbm.at[rows, :], out_sems.at[h])
            cp.start()
            outs.append(cp)
        for r in sends:
            r.wait_send()
        for r in a_sends:
            r.wait_send()
        for cp in outs:
            cp.wait()

    return pl.pallas_call(
        body,
        out_shape=jax.ShapeDtypeStruct((MB, NN), jnp.float32),
        in_specs=[
            pl.BlockSpec(memory_space=pl.ANY),
            pl.BlockSpec(memory_space=pl.ANY),
        ],
        out_specs=pl.BlockSpec(memory_space=pl.ANY),
        scratch_shapes=[
            pltpu.VMEM((P, MB, KB), jnp.float32),
            pltpu.VMEM((3, MB, KB), jnp.bfloat16),
            pltpu.VMEM((3, MB, KB), jnp.bfloat16),
            pltpu.VMEM((2, KB, NN), jnp.float32),
            pltpu.VMEM((MB, NN), jnp.float32),
            pltpu.VMEM((8, 128), jnp.float32),
            pltpu.VMEM((3, 8, 128), jnp.float32),
            pltpu.SemaphoreType.DMA((3, NC)),
            pltpu.SemaphoreType.DMA((3, NC)),
            pltpu.SemaphoreType.DMA((3,)),
            pltpu.SemaphoreType.DMA((3,)),
            pltpu.SemaphoreType.DMA((6,)),
            pltpu.SemaphoreType.DMA((NQ,)),
        ],
        compiler_params=pltpu.CompilerParams(
            collective_id=0,
            vmem_limit_bytes=60 * 1024 * 1024,
        ),
    )(x, w_mat)
